# Initial kernel scaffold; baseline (speedup 1.0000x reference)
import jax
import jax.numpy as jnp
from jax.experimental import pallas as pl


def kernel(x, adj_indices, weight, bias):
    raise NotImplementedError("write your pallas kernel here")


# speedup vs baseline: 34.7972x; 34.7972x over previous
import functools

import jax
import jax.numpy as jnp
from jax import lax
from jax.experimental import pallas as pl
from jax.experimental.pallas import tpu as pltpu


def _round_up(v, m):
    return ((v + m - 1) // m) * m


def _linear_kernel(w_ref, x_ref, b_ref, out_ref):
    out_ref[...] = jnp.dot(x_ref[...], w_ref[...],
                           preferred_element_type=jnp.float32,
                           precision=lax.Precision.HIGHEST) + b_ref[...]


def _att_kernel(c, eb, unroll, cchunk, table_ref, idx_ref, out_ref,
                xg_ref, yg_ref, idx_smem, sem):
    cp = pltpu.make_async_copy(idx_ref.at[0, 0], idx_smem, sem)
    cp.start()
    cp.wait()

    def gather_chunk(ci, carry):
        base = ci * unroll
        for j in range(unroll):
            e = base + j
            i0 = idx_smem[e]
            i1 = idx_smem[eb + e]
            xg_ref[e, 0] = table_ref[i0, 0]
            yg_ref[e, 0] = table_ref[i1, 0]
        return carry

    lax.fori_loop(0, eb // unroll, gather_chunk, 0)

    for ci in range(eb // cchunk):
        lo = ci * cchunk
        hi = lo + cchunk
        xg = xg_ref[lo:hi]
        yg = yg_ref[lo:hi]
        s = jnp.sum(xg * yg, axis=-1)
        sx = jnp.sum(xg * xg, axis=-1)
        sy = jnp.sum(yg * yg, axis=-1)
        hh = jnp.sqrt((sx + c) * (sy + c))
        res = jnp.clip(-(c + s - hh), 1e-10, 1.0)
        out_ref[lo:hi, :] = jnp.exp(-res)


def kernel(x, adj_indices, weight, bias):
    n, dim = x.shape
    d = dim - 1
    e = adj_indices.shape[1]
    c = 1.0

    nc = min(2048, _round_up(n, 8))
    n_pad = _round_up(n, nc)
    xf = x.astype(jnp.float32)
    if n_pad > n:
        xf = jnp.pad(xf, ((0, n_pad - n), (0, 0)))
    w_aug = jnp.concatenate(
        [jnp.zeros((1, d), jnp.float32), weight.astype(jnp.float32).T], axis=0)
    b_row = bias.astype(jnp.float32).reshape(1, d)

    table = pl.pallas_call(
        _linear_kernel,
        out_shape=jax.ShapeDtypeStruct((n_pad, d), jnp.float32),
        grid=(n_pad // nc,),
        in_specs=[
            pl.BlockSpec((dim, d), lambda i: (0, 0)),
            pl.BlockSpec((nc, dim), lambda i: (i, 0)),
            pl.BlockSpec((1, d), lambda i: (0, 0)),
        ],
        out_specs=pl.BlockSpec((nc, d), lambda i: (i, 0)),
        compiler_params=pltpu.CompilerParams(
            dimension_semantics=("parallel",)),
    )(w_aug, xf, b_row)

    table3 = table.reshape(n_pad, 1, d)

    eb = 8192
    e_pad = _round_up(e, eb)
    nb = e_pad // eb
    idx = adj_indices.astype(jnp.int32)
    pad = e_pad - e
    idx0 = jnp.pad(idx[0], (0, pad)).reshape(nb, 1, eb)
    idx1 = jnp.pad(idx[1], (0, pad)).reshape(nb, 1, eb)
    idx_cat = jnp.concatenate([idx0, idx1], axis=-1)

    att2 = pl.pallas_call(
        functools.partial(_att_kernel, c, eb, 32, 512),
        out_shape=jax.ShapeDtypeStruct((e_pad, 1), jnp.float32),
        grid=(nb,),
        in_specs=[
            pl.BlockSpec((n_pad, 1, d), lambda i: (0, 0, 0)),
            pl.BlockSpec((1, 1, 2 * eb), lambda i: (i, 0, 0)),
        ],
        out_specs=pl.BlockSpec((eb, 1), lambda i: (i, 0)),
        scratch_shapes=[
            pltpu.VMEM((eb, 1, d), jnp.float32),
            pltpu.VMEM((eb, 1, d), jnp.float32),
            pltpu.SMEM((2 * eb,), jnp.int32),
            pltpu.SemaphoreType.DMA,
        ],
        compiler_params=pltpu.CompilerParams(
            dimension_semantics=("parallel",),
            vmem_limit_bytes=60 * 1024 * 1024),
    )(table3, idx_cat)

    return att2[:e, 0]


# speedup vs baseline: 88.3398x; 2.5387x over previous
import functools

import jax
import jax.numpy as jnp
from jax import lax
from jax.experimental import pallas as pl
from jax.experimental.pallas import tpu as pltpu


def _round_up(v, m):
    return ((v + m - 1) // m) * m


def _linear_kernel(w_ref, x_ref, b_ref, out_ref):
    out_ref[...] = jnp.dot(x_ref[...], w_ref[...],
                           preferred_element_type=jnp.float32,
                           precision=lax.Precision.HIGHEST) + b_ref[...]


def _att_kernel(c, eb, unroll, cchunk, table_ref, idx_ref, out_ref,
                xg_ref, yg_ref, xg2_ref, yg2_ref, idx_smem, sem):
    d = table_ref.shape[-1]
    cp = pltpu.make_async_copy(idx_ref.at[0, 0], idx_smem, sem)
    cp.start()
    cp.wait()

    def gather_chunk(ci, carry):
        base = ci * unroll
        for j in range(unroll):
            e = base + j
            i0 = idx_smem[e]
            i1 = idx_smem[eb + e]
            xg_ref[e, 0] = table_ref[i0, 0]
            yg_ref[e, 0] = table_ref[i1, 0]
        return carry

    lax.fori_loop(0, eb // unroll, gather_chunk, 0)

    xg2_ref[...] = xg_ref[...].reshape(eb, d)
    yg2_ref[...] = yg_ref[...].reshape(eb, d)

    for ci in range(eb // cchunk):
        lo = ci * cchunk
        hi = lo + cchunk
        xg = xg2_ref[lo:hi, :]
        yg = yg2_ref[lo:hi, :]
        s = jnp.sum(xg * yg, axis=-1, keepdims=True)
        sx = jnp.sum(xg * xg, axis=-1, keepdims=True)
        sy = jnp.sum(yg * yg, axis=-1, keepdims=True)
        hh = jnp.sqrt((sx + c) * (sy + c))
        res = jnp.clip(-(c + s - hh), 1e-10, 1.0)
        out_ref[lo:hi, :] = jnp.exp(-res)


def kernel(x, adj_indices, weight, bias):
    n, dim = x.shape
    d = dim - 1
    e = adj_indices.shape[1]
    c = 1.0

    nc = min(2048, _round_up(n, 8))
    n_pad = _round_up(n, nc)
    xf = x.astype(jnp.float32)
    if n_pad > n:
        xf = jnp.pad(xf, ((0, n_pad - n), (0, 0)))
    w_aug = jnp.concatenate(
        [jnp.zeros((1, d), jnp.float32), weight.astype(jnp.float32).T], axis=0)
    b_row = bias.astype(jnp.float32).reshape(1, d)

    table = pl.pallas_call(
        _linear_kernel,
        out_shape=jax.ShapeDtypeStruct((n_pad, d), jnp.float32),
        grid=(n_pad // nc,),
        in_specs=[
            pl.BlockSpec((dim, d), lambda i: (0, 0)),
            pl.BlockSpec((nc, dim), lambda i: (i, 0)),
            pl.BlockSpec((1, d), lambda i: (0, 0)),
        ],
        out_specs=pl.BlockSpec((nc, d), lambda i: (i, 0)),
        compiler_params=pltpu.CompilerParams(
            dimension_semantics=("parallel",)),
    )(w_aug, xf, b_row)

    table3 = table.reshape(n_pad, 1, d)

    eb = 4096
    e_pad = _round_up(e, eb)
    nb = e_pad // eb
    idx = adj_indices.astype(jnp.int32)
    pad = e_pad - e
    idx0 = jnp.pad(idx[0], (0, pad)).reshape(nb, 1, eb)
    idx1 = jnp.pad(idx[1], (0, pad)).reshape(nb, 1, eb)
    idx_cat = jnp.concatenate([idx0, idx1], axis=-1)

    att2 = pl.pallas_call(
        functools.partial(_att_kernel, c, eb, 32, 512),
        out_shape=jax.ShapeDtypeStruct((e_pad, 1), jnp.float32),
        grid=(nb,),
        in_specs=[
            pl.BlockSpec((n_pad, 1, d), lambda i: (0, 0, 0)),
            pl.BlockSpec((1, 1, 2 * eb), lambda i: (i, 0, 0)),
        ],
        out_specs=pl.BlockSpec((eb, 1), lambda i: (i, 0)),
        scratch_shapes=[
            pltpu.VMEM((eb, 1, d), jnp.float32),
            pltpu.VMEM((eb, 1, d), jnp.float32),
            pltpu.VMEM((eb, d), jnp.float32),
            pltpu.VMEM((eb, d), jnp.float32),
            pltpu.SMEM((2 * eb,), jnp.int32),
            pltpu.SemaphoreType.DMA,
        ],
        compiler_params=pltpu.CompilerParams(
            dimension_semantics=("parallel",),
            vmem_limit_bytes=60 * 1024 * 1024),
    )(table3, idx_cat)

    return att2[:e, 0]


# speedup vs baseline: 90.7512x; 1.0273x over previous
import functools

import jax
import jax.numpy as jnp
from jax import lax
from jax.experimental import pallas as pl
from jax.experimental.pallas import tpu as pltpu


def _round_up(v, m):
    return ((v + m - 1) // m) * m


def _linear_kernel(w_ref, x_ref, b_ref, out_ref):
    out_ref[...] = jnp.dot(x_ref[...], w_ref[...],
                           preferred_element_type=jnp.float32,
                           precision=lax.Precision.HIGHEST) + b_ref[...]


def _att_kernel(c, eb, unroll, cchunk, packed, table_ref, idx_ref, out_ref,
                xg_ref, yg_ref, xg2_ref, yg2_ref, idx_smem, sem):
    d = table_ref.shape[-1]
    cp = pltpu.make_async_copy(idx_ref.at[0, 0], idx_smem, sem)
    cp.start()
    cp.wait()

    def gather_chunk(ci, carry):
        base = ci * unroll
        for j in range(unroll):
            e = base + j
            if packed:
                p = idx_smem[e]
                i0 = jnp.bitwise_and(p, 0xFFFF)
                i1 = lax.shift_right_logical(p, 16)
            else:
                i0 = idx_smem[e]
                i1 = idx_smem[eb + e]
            xg_ref[e, 0] = table_ref[i0, 0]
            yg_ref[e, 0] = table_ref[i1, 0]
        return carry

    lax.fori_loop(0, eb // unroll, gather_chunk, 0)

    xg2_ref[...] = xg_ref[...].reshape(eb, d)
    yg2_ref[...] = yg_ref[...].reshape(eb, d)

    for ci in range(eb // cchunk):
        lo = ci * cchunk
        hi = lo + cchunk
        xg = xg2_ref[lo:hi, :]
        yg = yg2_ref[lo:hi, :]
        s = jnp.sum(xg * yg, axis=-1, keepdims=True)
        sx = jnp.sum(xg * xg, axis=-1, keepdims=True)
        sy = jnp.sum(yg * yg, axis=-1, keepdims=True)
        hh = jnp.sqrt((sx + c) * (sy + c))
        res = jnp.clip(-(c + s - hh), 1e-10, 1.0)
        out_ref[lo:hi, :] = jnp.exp(-res)


def kernel(x, adj_indices, weight, bias):
    n, dim = x.shape
    d = dim - 1
    e = adj_indices.shape[1]
    c = 1.0

    nc = min(2048, _round_up(n, 8))
    n_pad = _round_up(n, nc)
    xf = x.astype(jnp.float32)
    if n_pad > n:
        xf = jnp.pad(xf, ((0, n_pad - n), (0, 0)))
    w_aug = jnp.concatenate(
        [jnp.zeros((1, d), jnp.float32), weight.astype(jnp.float32).T], axis=0)
    b_row = bias.astype(jnp.float32).reshape(1, d)

    table = pl.pallas_call(
        _linear_kernel,
        out_shape=jax.ShapeDtypeStruct((n_pad, d), jnp.float32),
        grid=(n_pad // nc,),
        in_specs=[
            pl.BlockSpec((dim, d), lambda i: (0, 0)),
            pl.BlockSpec((nc, dim), lambda i: (i, 0)),
            pl.BlockSpec((1, d), lambda i: (0, 0)),
        ],
        out_specs=pl.BlockSpec((nc, d), lambda i: (i, 0)),
        compiler_params=pltpu.CompilerParams(
            dimension_semantics=("parallel",)),
    )(w_aug, xf, b_row)

    table3 = table.reshape(n_pad, 1, d)

    eb = 4096
    e_pad = _round_up(e, eb)
    nb = e_pad // eb
    idx = adj_indices.astype(jnp.int32)
    pad = e_pad - e
    idx0 = jnp.pad(idx[0], (0, pad))
    idx1 = jnp.pad(idx[1], (0, pad))
    packed = n_pad <= (1 << 16)
    if packed:
        idx_cat = jnp.bitwise_or(
            idx0, jnp.left_shift(idx1, 16)).reshape(nb, 1, eb)
        idx_w = eb
    else:
        idx_cat = jnp.concatenate(
            [idx0.reshape(nb, 1, eb), idx1.reshape(nb, 1, eb)], axis=-1)
        idx_w = 2 * eb

    att2 = pl.pallas_call(
        functools.partial(_att_kernel, c, eb, 64, 512, packed),
        out_shape=jax.ShapeDtypeStruct((e_pad, 1), jnp.float32),
        grid=(nb,),
        in_specs=[
            pl.BlockSpec((n_pad, 1, d), lambda i: (0, 0, 0)),
            pl.BlockSpec((1, 1, idx_w), lambda i: (i, 0, 0)),
        ],
        out_specs=pl.BlockSpec((eb, 1), lambda i: (i, 0)),
        scratch_shapes=[
            pltpu.VMEM((eb, 1, d), jnp.float32),
            pltpu.VMEM((eb, 1, d), jnp.float32),
            pltpu.VMEM((eb, d), jnp.float32),
            pltpu.VMEM((eb, d), jnp.float32),
            pltpu.SMEM((idx_w,), jnp.int32),
            pltpu.SemaphoreType.DMA,
        ],
        compiler_params=pltpu.CompilerParams(
            dimension_semantics=("parallel",),
            vmem_limit_bytes=60 * 1024 * 1024),
    )(table3, idx_cat)

    return att2[:e, 0]


# speedup vs baseline: 92.8106x; 1.0227x over previous
import functools

import jax
import jax.numpy as jnp
from jax import lax
from jax.experimental import pallas as pl
from jax.experimental.pallas import tpu as pltpu


def _round_up(v, m):
    return ((v + m - 1) // m) * m


def _linear_kernel(w_ref, x_ref, b_ref, out_ref):
    out_ref[...] = jnp.dot(x_ref[...], w_ref[...],
                           preferred_element_type=jnp.float32,
                           precision=lax.Precision.HIGHEST) + b_ref[...]


def _att_kernel(c, eb, unroll, cchunk, packed, table_ref, idx_ref, out_ref,
                xg_ref, yg_ref, xg2_ref, yg2_ref, idx_smem, sem):
    d = table_ref.shape[-1]
    cp = pltpu.make_async_copy(idx_ref.at[0, 0], idx_smem, sem)
    cp.start()
    cp.wait()

    def gather_chunk(ci, carry):
        base = ci * unroll
        for j in range(unroll):
            e = base + j
            if packed:
                p = idx_smem[e]
                i0 = jnp.bitwise_and(p, 0xFFFF)
                i1 = lax.shift_right_logical(p, 16)
            else:
                i0 = idx_smem[e]
                i1 = idx_smem[eb + e]
            xg_ref[e, 0] = table_ref[i0, 0]
            yg_ref[e, 0] = table_ref[i1, 0]
        return carry

    lax.fori_loop(0, eb // unroll, gather_chunk, 0)

    xg2_ref[...] = xg_ref[...].reshape(eb, d)
    yg2_ref[...] = yg_ref[...].reshape(eb, d)

    for ci in range(eb // cchunk):
        lo = ci * cchunk
        hi = lo + cchunk
        xg = xg2_ref[lo:hi, :]
        yg = yg2_ref[lo:hi, :]
        s = jnp.sum(xg * yg, axis=-1, keepdims=True)
        sx = jnp.sum(xg * xg, axis=-1, keepdims=True)
        sy = jnp.sum(yg * yg, axis=-1, keepdims=True)
        hh = jnp.sqrt((sx + c) * (sy + c))
        res = jnp.clip(-(c + s - hh), 1e-10, 1.0)
        out_ref[lo:hi, :] = jnp.exp(-res)


def kernel(x, adj_indices, weight, bias):
    n, dim = x.shape
    d = dim - 1
    e = adj_indices.shape[1]
    c = 1.0

    nc = min(2048, _round_up(n, 8))
    n_pad = _round_up(n, nc)
    xf = x.astype(jnp.float32)
    if n_pad > n:
        xf = jnp.pad(xf, ((0, n_pad - n), (0, 0)))
    w_aug = jnp.concatenate(
        [jnp.zeros((1, d), jnp.float32), weight.astype(jnp.float32).T], axis=0)
    b_row = bias.astype(jnp.float32).reshape(1, d)

    table = pl.pallas_call(
        _linear_kernel,
        out_shape=jax.ShapeDtypeStruct((n_pad, d), jnp.float32),
        grid=(n_pad // nc,),
        in_specs=[
            pl.BlockSpec((dim, d), lambda i: (0, 0)),
            pl.BlockSpec((nc, dim), lambda i: (i, 0)),
            pl.BlockSpec((1, d), lambda i: (0, 0)),
        ],
        out_specs=pl.BlockSpec((nc, d), lambda i: (i, 0)),
        compiler_params=pltpu.CompilerParams(
            dimension_semantics=("parallel",)),
    )(w_aug, xf, b_row)

    table3 = table.reshape(n_pad, 1, d)

    eb = 8192
    e_pad = _round_up(e, eb)
    nb = e_pad // eb
    idx = adj_indices.astype(jnp.int32)
    pad = e_pad - e
    idx0 = jnp.pad(idx[0], (0, pad))
    idx1 = jnp.pad(idx[1], (0, pad))
    packed = n_pad <= (1 << 16)
    if packed:
        idx_cat = jnp.bitwise_or(
            idx0, jnp.left_shift(idx1, 16)).reshape(nb, 1, eb)
        idx_w = eb
    else:
        idx_cat = jnp.concatenate(
            [idx0.reshape(nb, 1, eb), idx1.reshape(nb, 1, eb)], axis=-1)
        idx_w = 2 * eb

    att2 = pl.pallas_call(
        functools.partial(_att_kernel, c, eb, 64, 512, packed),
        out_shape=jax.ShapeDtypeStruct((e_pad, 1), jnp.float32),
        grid=(nb,),
        in_specs=[
            pl.BlockSpec((n_pad, 1, d), lambda i: (0, 0, 0)),
            pl.BlockSpec((1, 1, idx_w), lambda i: (i, 0, 0)),
        ],
        out_specs=pl.BlockSpec((eb, 1), lambda i: (i, 0)),
        scratch_shapes=[
            pltpu.VMEM((eb, 1, d), jnp.float32),
            pltpu.VMEM((eb, 1, d), jnp.float32),
            pltpu.VMEM((eb, d), jnp.float32),
            pltpu.VMEM((eb, d), jnp.float32),
            pltpu.SMEM((idx_w,), jnp.int32),
            pltpu.SemaphoreType.DMA,
        ],
        compiler_params=pltpu.CompilerParams(
            dimension_semantics=("parallel",),
            vmem_limit_bytes=62 * 1024 * 1024),
    )(table3, idx_cat)

    return att2[:e, 0]


# speedup vs baseline: 95.1458x; 1.0252x over previous
import functools

import jax
import jax.numpy as jnp
from jax import lax
from jax.experimental import pallas as pl
from jax.experimental.pallas import tpu as pltpu


def _round_up(v, m):
    return ((v + m - 1) // m) * m


def _linear_kernel(w_ref, x_ref, b_ref, out_ref):
    out_ref[...] = jnp.dot(x_ref[...], w_ref[...],
                           preferred_element_type=jnp.float32,
                           precision=lax.Precision.HIGHEST) + b_ref[...]


def _att_kernel(c, eb, unroll, cchunk, packed, table_ref, idx_ref, out_ref,
                xg_ref, yg_ref, xg2_ref, yg2_ref, idx_smem, sem):
    d = table_ref.shape[-1]
    cp = pltpu.make_async_copy(idx_ref.at[0, 0], idx_smem, sem)
    cp.start()
    cp.wait()

    def gather_chunk(ci, carry):
        base = ci * unroll
        for j in range(unroll):
            e = base + j
            if packed:
                p = idx_smem[e]
                i0 = jnp.bitwise_and(p, 0xFFFF)
                i1 = lax.shift_right_logical(p, 16)
            else:
                i0 = idx_smem[e]
                i1 = idx_smem[eb + e]
            xg_ref[e, 0] = table_ref[i0, 0]
            yg_ref[e, 0] = table_ref[i1, 0]
        return carry

    lax.fori_loop(0, eb // unroll, gather_chunk, 0)

    xg2_ref[...] = xg_ref[...].reshape(eb, d)
    yg2_ref[...] = yg_ref[...].reshape(eb, d)

    for ci in range(eb // cchunk):
        lo = ci * cchunk
        hi = lo + cchunk
        xg = xg2_ref[lo:hi, :]
        yg = yg2_ref[lo:hi, :]
        s = jnp.sum(xg * yg, axis=-1, keepdims=True)
        sx = jnp.sum(xg * xg, axis=-1, keepdims=True)
        sy = jnp.sum(yg * yg, axis=-1, keepdims=True)
        hh = jnp.sqrt((sx + c) * (sy + c))
        res = jnp.clip(-(c + s - hh), 1e-10, 1.0)
        out_ref[lo:hi, :] = jnp.exp(-res)


def kernel(x, adj_indices, weight, bias):
    n, dim = x.shape
    d = dim - 1
    e = adj_indices.shape[1]
    c = 1.0

    nc = min(2048, _round_up(n, 8))
    n_pad = _round_up(n, nc)
    xf = x.astype(jnp.float32)
    if n_pad > n:
        xf = jnp.pad(xf, ((0, n_pad - n), (0, 0)))
    w_aug = jnp.concatenate(
        [jnp.zeros((1, d), jnp.float32), weight.astype(jnp.float32).T], axis=0)
    b_row = bias.astype(jnp.float32).reshape(1, d)

    table = pl.pallas_call(
        _linear_kernel,
        out_shape=jax.ShapeDtypeStruct((n_pad, d), jnp.float32),
        grid=(n_pad // nc,),
        in_specs=[
            pl.BlockSpec((dim, d), lambda i: (0, 0)),
            pl.BlockSpec((nc, dim), lambda i: (i, 0)),
            pl.BlockSpec((1, d), lambda i: (0, 0)),
        ],
        out_specs=pl.BlockSpec((nc, d), lambda i: (i, 0)),
        compiler_params=pltpu.CompilerParams(
            dimension_semantics=("parallel",)),
    )(w_aug, xf, b_row)

    table3 = table.reshape(n_pad, 1, d)

    eb = 8192
    e_pad = _round_up(e, eb)
    nb = e_pad // eb
    idx = adj_indices.astype(jnp.int32)
    pad = e_pad - e
    idx0 = jnp.pad(idx[0], (0, pad))
    idx1 = jnp.pad(idx[1], (0, pad))
    packed = n_pad <= (1 << 16)
    if packed:
        idx_cat = jnp.bitwise_or(
            idx0, jnp.left_shift(idx1, 16)).reshape(nb, 1, eb)
        idx_w = eb
    else:
        idx_cat = jnp.concatenate(
            [idx0.reshape(nb, 1, eb), idx1.reshape(nb, 1, eb)], axis=-1)
        idx_w = 2 * eb

    att2 = pl.pallas_call(
        functools.partial(_att_kernel, c, eb, 64, 128, packed),
        out_shape=jax.ShapeDtypeStruct((e_pad, 1), jnp.float32),
        grid=(nb,),
        in_specs=[
            pl.BlockSpec((n_pad, 1, d), lambda i: (0, 0, 0)),
            pl.BlockSpec((1, 1, idx_w), lambda i: (i, 0, 0)),
        ],
        out_specs=pl.BlockSpec((eb, 1), lambda i: (i, 0)),
        scratch_shapes=[
            pltpu.VMEM((eb, 1, d), jnp.float32),
            pltpu.VMEM((eb, 1, d), jnp.float32),
            pltpu.VMEM((eb, d), jnp.float32),
            pltpu.VMEM((eb, d), jnp.float32),
            pltpu.SMEM((idx_w,), jnp.int32),
            pltpu.SemaphoreType.DMA,
        ],
        compiler_params=pltpu.CompilerParams(
            dimension_semantics=("parallel",),
            vmem_limit_bytes=62 * 1024 * 1024),
    )(table3, idx_cat)

    return att2[:e, 0]


# speedup vs baseline: 95.9555x; 1.0085x over previous
import functools

import jax
import jax.numpy as jnp
from jax import lax
from jax.experimental import pallas as pl
from jax.experimental.pallas import tpu as pltpu


def _round_up(v, m):
    return ((v + m - 1) // m) * m


def _linear_kernel(w_ref, x_ref, b_ref, out_ref):
    out_ref[...] = jnp.dot(x_ref[...], w_ref[...],
                           preferred_element_type=jnp.float32,
                           precision=lax.Precision.HIGHEST) + b_ref[...]


def _att_kernel(c, eb, unroll, cchunk, packed, table_ref, idx_ref, out_ref,
                xg_ref, yg_ref, xg2_ref, yg2_ref, idx_smem, sem):
    d = table_ref.shape[-1]
    cp = pltpu.make_async_copy(idx_ref.at[0, 0], idx_smem, sem)
    cp.start()
    cp.wait()

    def gather_chunk(ci, carry):
        base = ci * unroll
        for j in range(unroll):
            e = base + j
            if packed:
                p = idx_smem[e]
                i0 = jnp.bitwise_and(p, 0xFFFF)
                i1 = lax.shift_right_logical(p, 16)
            else:
                i0 = idx_smem[e]
                i1 = idx_smem[eb + e]
            xg_ref[e, 0] = table_ref[i0, 0]
            yg_ref[e, 0] = table_ref[i1, 0]
        return carry

    lax.fori_loop(0, eb // unroll, gather_chunk, 0)

    xg2_ref[...] = xg_ref[...].reshape(eb, d)
    yg2_ref[...] = yg_ref[...].reshape(eb, d)

    for ci in range(eb // cchunk):
        lo = ci * cchunk
        hi = lo + cchunk
        xg = xg2_ref[lo:hi, :]
        yg = yg2_ref[lo:hi, :]
        s = jnp.sum(xg * yg, axis=-1, keepdims=True)
        sx = jnp.sum(xg * xg, axis=-1, keepdims=True)
        sy = jnp.sum(yg * yg, axis=-1, keepdims=True)
        hh = jnp.sqrt((sx + c) * (sy + c))
        res = jnp.clip(-(c + s - hh), 1e-10, 1.0)
        out_ref[lo:hi, :] = jnp.exp(-res)


def kernel(x, adj_indices, weight, bias):
    n, dim = x.shape
    d = dim - 1
    e = adj_indices.shape[1]
    c = 1.0

    nc = min(2048, _round_up(n, 8))
    n_pad = _round_up(n, nc)
    xf = x.astype(jnp.float32)
    if n_pad > n:
        xf = jnp.pad(xf, ((0, n_pad - n), (0, 0)))
    w_aug = jnp.concatenate(
        [jnp.zeros((1, d), jnp.float32), weight.astype(jnp.float32).T], axis=0)
    b_row = bias.astype(jnp.float32).reshape(1, d)

    table = pl.pallas_call(
        _linear_kernel,
        out_shape=jax.ShapeDtypeStruct((n_pad, d), jnp.float32),
        grid=(n_pad // nc,),
        in_specs=[
            pl.BlockSpec((dim, d), lambda i: (0, 0)),
            pl.BlockSpec((nc, dim), lambda i: (i, 0)),
            pl.BlockSpec((1, d), lambda i: (0, 0)),
        ],
        out_specs=pl.BlockSpec((nc, d), lambda i: (i, 0)),
        compiler_params=pltpu.CompilerParams(
            dimension_semantics=("parallel",)),
    )(w_aug, xf, b_row)

    table3 = table.reshape(n_pad, 1, d)

    eb = 8192
    e_pad = _round_up(e, eb)
    nb = e_pad // eb
    idx = adj_indices.astype(jnp.int32)
    pad = e_pad - e
    idx0 = jnp.pad(idx[0], (0, pad))
    idx1 = jnp.pad(idx[1], (0, pad))
    packed = n_pad <= (1 << 16)
    if packed:
        idx_cat = jnp.bitwise_or(
            idx0, jnp.left_shift(idx1, 16)).reshape(nb, 1, eb)
        idx_w = eb
    else:
        idx_cat = jnp.concatenate(
            [idx0.reshape(nb, 1, eb), idx1.reshape(nb, 1, eb)], axis=-1)
        idx_w = 2 * eb

    att2 = pl.pallas_call(
        functools.partial(_att_kernel, c, eb, 128, 128, packed),
        out_shape=jax.ShapeDtypeStruct((e_pad, 1), jnp.float32),
        grid=(nb,),
        in_specs=[
            pl.BlockSpec((n_pad, 1, d), lambda i: (0, 0, 0)),
            pl.BlockSpec((1, 1, idx_w), lambda i: (i, 0, 0)),
        ],
        out_specs=pl.BlockSpec((eb, 1), lambda i: (i, 0)),
        scratch_shapes=[
            pltpu.VMEM((eb, 1, d), jnp.float32),
            pltpu.VMEM((eb, 1, d), jnp.float32),
            pltpu.VMEM((eb, d), jnp.float32),
            pltpu.VMEM((eb, d), jnp.float32),
            pltpu.SMEM((idx_w,), jnp.int32),
            pltpu.SemaphoreType.DMA,
        ],
        compiler_params=pltpu.CompilerParams(
            dimension_semantics=("parallel",),
            vmem_limit_bytes=62 * 1024 * 1024),
    )(table3, idx_cat)

    return att2[:e, 0]


# speedup vs baseline: 96.7011x; 1.0078x over previous
import functools

import jax
import jax.numpy as jnp
from jax import lax
from jax.experimental import pallas as pl
from jax.experimental.pallas import tpu as pltpu


def _round_up(v, m):
    return ((v + m - 1) // m) * m


def _linear_kernel(w_ref, x_ref, b_ref, out_ref):
    out_ref[...] = jnp.dot(x_ref[...], w_ref[...],
                           preferred_element_type=jnp.float32,
                           precision=lax.Precision.HIGHEST) + b_ref[...]


def _att_kernel(c, eb, unroll, cchunk, packed, table_ref, idx_ref, out_ref,
                xg_ref, yg_ref, xg2_ref, yg2_ref, idx_smem, sem, sem2):
    d = table_ref.shape[-1]
    iw = idx_smem.shape[0]
    h = iw // 2
    cp_a = pltpu.make_async_copy(idx_ref.at[0, 0, pl.ds(0, h)],
                                 idx_smem.at[pl.ds(0, h)], sem)
    cp_b = pltpu.make_async_copy(idx_ref.at[0, 0, pl.ds(h, h)],
                                 idx_smem.at[pl.ds(h, h)], sem2)
    cp_a.start()
    cp_b.start()

    def gather_chunk(ci, carry):
        base = ci * unroll
        for j in range(unroll):
            e = base + j
            if packed:
                p = idx_smem[e]
                i0 = jnp.bitwise_and(p, 0xFFFF)
                i1 = lax.shift_right_logical(p, 16)
            else:
                i0 = idx_smem[e]
                i1 = idx_smem[eb + e]
            xg_ref[e, 0] = table_ref[i0, 0]
            yg_ref[e, 0] = table_ref[i1, 0]
        return carry

    ntrips = eb // unroll
    if packed:
        cp_a.wait()
        lax.fori_loop(0, ntrips // 2, gather_chunk, 0)
        cp_b.wait()
        lax.fori_loop(ntrips // 2, ntrips, gather_chunk, 0)
    else:
        cp_a.wait()
        cp_b.wait()
        lax.fori_loop(0, ntrips, gather_chunk, 0)

    xg2_ref[...] = xg_ref[...].reshape(eb, d)
    yg2_ref[...] = yg_ref[...].reshape(eb, d)

    for ci in range(eb // cchunk):
        lo = ci * cchunk
        hi = lo + cchunk
        xg = xg2_ref[lo:hi, :]
        yg = yg2_ref[lo:hi, :]
        s = jnp.sum(xg * yg, axis=-1, keepdims=True)
        sx = jnp.sum(xg * xg, axis=-1, keepdims=True)
        sy = jnp.sum(yg * yg, axis=-1, keepdims=True)
        hh = jnp.sqrt((sx + c) * (sy + c))
        res = jnp.clip(-(c + s - hh), 1e-10, 1.0)
        out_ref[lo:hi, :] = jnp.exp(-res)


def kernel(x, adj_indices, weight, bias):
    n, dim = x.shape
    d = dim - 1
    e = adj_indices.shape[1]
    c = 1.0

    nc = min(2048, _round_up(n, 8))
    n_pad = _round_up(n, nc)
    xf = x.astype(jnp.float32)
    if n_pad > n:
        xf = jnp.pad(xf, ((0, n_pad - n), (0, 0)))
    w_aug = jnp.concatenate(
        [jnp.zeros((1, d), jnp.float32), weight.astype(jnp.float32).T], axis=0)
    b_row = bias.astype(jnp.float32).reshape(1, d)

    table = pl.pallas_call(
        _linear_kernel,
        out_shape=jax.ShapeDtypeStruct((n_pad, d), jnp.float32),
        grid=(n_pad // nc,),
        in_specs=[
            pl.BlockSpec((dim, d), lambda i: (0, 0)),
            pl.BlockSpec((nc, dim), lambda i: (i, 0)),
            pl.BlockSpec((1, d), lambda i: (0, 0)),
        ],
        out_specs=pl.BlockSpec((nc, d), lambda i: (i, 0)),
        compiler_params=pltpu.CompilerParams(
            dimension_semantics=("parallel",)),
    )(w_aug, xf, b_row)

    table3 = table.reshape(n_pad, 1, d)

    eb = 8192
    e_pad = _round_up(e, eb)
    nb = e_pad // eb
    idx = adj_indices.astype(jnp.int32)
    pad = e_pad - e
    idx0 = jnp.pad(idx[0], (0, pad))
    idx1 = jnp.pad(idx[1], (0, pad))
    packed = n_pad <= (1 << 16)
    if packed:
        idx_cat = jnp.bitwise_or(
            idx0, jnp.left_shift(idx1, 16)).reshape(nb, 1, eb)
        idx_w = eb
    else:
        idx_cat = jnp.concatenate(
            [idx0.reshape(nb, 1, eb), idx1.reshape(nb, 1, eb)], axis=-1)
        idx_w = 2 * eb

    att2 = pl.pallas_call(
        functools.partial(_att_kernel, c, eb, 128, 128, packed),
        out_shape=jax.ShapeDtypeStruct((e_pad, 1), jnp.float32),
        grid=(nb,),
        in_specs=[
            pl.BlockSpec((n_pad, 1, d), lambda i: (0, 0, 0)),
            pl.BlockSpec((1, 1, idx_w), lambda i: (i, 0, 0)),
        ],
        out_specs=pl.BlockSpec((eb, 1), lambda i: (i, 0)),
        scratch_shapes=[
            pltpu.VMEM((eb, 1, d), jnp.float32),
            pltpu.VMEM((eb, 1, d), jnp.float32),
            pltpu.VMEM((eb, d), jnp.float32),
            pltpu.VMEM((eb, d), jnp.float32),
            pltpu.SMEM((idx_w,), jnp.int32),
            pltpu.SemaphoreType.DMA,
            pltpu.SemaphoreType.DMA,
        ],
        compiler_params=pltpu.CompilerParams(
            dimension_semantics=("parallel",),
            vmem_limit_bytes=62 * 1024 * 1024),
    )(table3, idx_cat)

    return att2[:e, 0]
